# baseline (speedup 1.0000x reference)
import functools

import jax
import jax.numpy as jnp
from jax import lax
from jax.experimental import pallas as pl
from jax.experimental.pallas import tpu as pltpu
from jax.experimental.pallas import tpu_sc as plsc

D = 32
NCODE = 8192
NTOK = 8192
TB = 1024
KB = 2048
NKB = NCODE // KB


def _argmin_body(z_ref, zsq_ref, esq_ref, e_ref, mask_ref,
                 idx_ref, se_ref, ms_ref):
    zb = z_ref[...]
    zsq = zsq_ref[...]

    def chunk(k, carry):
        run_min, run_idx, run_exact = carry
        e_chunk = e_ref[pl.ds(k * KB, KB), :]
        esq = esq_ref[:, pl.ds(k * KB, KB)]
        mm = lax.dot_general(zb, e_chunk, (((1,), (1,)), ((), ())),
                             preferred_element_type=jnp.float32)
        d = (zsq + esq) - 2.0 * mm
        m = jnp.min(d, axis=1, keepdims=True)
        col = lax.broadcasted_iota(jnp.int32, (TB, KB), 1) + k * KB
        li = jnp.min(jnp.where(d == m, col, NCODE), axis=1, keepdims=True)
        better = m < run_min
        m_carry = m.astype(jnp.bfloat16).astype(jnp.float32)
        return (jnp.where(better, m_carry, run_min),
                jnp.where(better, li, run_idx),
                jnp.where(better, m, run_exact))

    init = (jnp.full((TB, 1), jnp.inf, jnp.float32),
            jnp.zeros((TB, 1), jnp.int32),
            jnp.full((TB, 1), jnp.inf, jnp.float32))
    run_min, run_idx, run_exact = lax.fori_loop(0, NKB, chunk, init)
    idx_ref[...] = run_idx
    mb = mask_ref[...]
    i = pl.program_id(0)
    se_ref[pl.ds(i, 1), :] = jnp.sum(run_exact * mb).reshape(1, 1)
    ms_ref[pl.ds(i, 1), :] = jnp.sum(mb).reshape(1, 1)


def _argmin_call(z_flat, z_sq, e_sq, embeddings, mask_flat):
    nblk = NTOK // TB
    return pl.pallas_call(
        _argmin_body,
        grid=(nblk,),
        in_specs=[
            pl.BlockSpec((TB, D), lambda i: (i, 0)),
            pl.BlockSpec((TB, 1), lambda i: (i, 0)),
            pl.BlockSpec((1, NCODE), lambda i: (0, 0)),
            pl.BlockSpec((NCODE, D), lambda i: (0, 0)),
            pl.BlockSpec((TB, 1), lambda i: (i, 0)),
        ],
        out_specs=[
            pl.BlockSpec((TB, 1), lambda i: (i, 0)),
            pl.BlockSpec((nblk, 1), lambda i: (0, 0)),
            pl.BlockSpec((nblk, 1), lambda i: (0, 0)),
        ],
        out_shape=[
            jax.ShapeDtypeStruct((NTOK, 1), jnp.int32),
            jax.ShapeDtypeStruct((nblk, 1), jnp.float32),
            jax.ShapeDtypeStruct((nblk, 1), jnp.float32),
        ],
    )(z_flat, z_sq, e_sq, embeddings, mask_flat)


GROW = 128


def _make_sc_gather():
    info = plsc.get_sparse_core_info()
    nw = info.num_cores * info.num_subcores
    b_per_w = NTOK // nw
    nch = b_per_w // 128
    mesh = plsc.VectorSubcoreMesh(core_axis_name="c", subcore_axis_name="s")

    @functools.partial(
        pl.kernel, mesh=mesh,
        out_type=jax.ShapeDtypeStruct((NTOK, GROW), jnp.float32),
        scratch_types=[
            pltpu.VMEM((nch, 128), jnp.int32),
            pltpu.VMEM((b_per_w, GROW), jnp.float32),
            pltpu.SemaphoreType.DMA,
        ],
    )
    def gather(table_hbm, idx_hbm, out_hbm, idx_v, rows_v, sem):
        wid = lax.axis_index("s") * info.num_cores + lax.axis_index("c")
        base = wid * b_per_w
        pltpu.sync_copy(idx_hbm.at[pl.ds(wid * nch, nch)], idx_v)
        for c in range(nch):
            pltpu.async_copy(
                table_hbm.at[idx_v.at[c]],
                rows_v.at[pl.ds(c * 128, 128)], sem).wait()
        pltpu.sync_copy(rows_v, out_hbm.at[pl.ds(base, b_per_w)])

    return gather


def kernel(z, mask, embeddings):
    orig_shape = z.shape
    z_flat = z.reshape(-1, z.shape[-1])
    z_sq = jnp.sum(z_flat ** 2, axis=1, keepdims=True)
    e_sq = jnp.sum(embeddings ** 2, axis=1).reshape(1, NCODE)
    mask_flat = mask.reshape(NTOK, 1)

    idx2d, se_parts, ms_parts = _argmin_call(
        z_flat, z_sq, e_sq, embeddings, mask_flat)
    idx = idx2d.reshape(NTOK)

    e_pad = jnp.pad(embeddings, ((0, 0), (0, GROW - D)))
    rows = _make_sc_gather()(e_pad, idx.reshape(NTOK // 128, 128))
    quantized = rows[:, :D].reshape(orig_shape)
    quantized_st = z + (quantized - z)

    mse = (jnp.sum(se_parts) / 32.0) / jnp.maximum(jnp.sum(ms_parts), 1.0)
    loss = mse + 0.25 * mse

    return quantized_st, idx.reshape(orig_shape[:-1]), loss


# speedup vs baseline: 1.0179x; 1.0179x over previous
import functools

import jax
import jax.numpy as jnp
from jax import lax
from jax.experimental import pallas as pl
from jax.experimental.pallas import tpu as pltpu
from jax.experimental.pallas import tpu_sc as plsc

D = 32
NCODE = 8192
NTOK = 8192
TB = 2048
KB = 2048
NKB = NCODE // KB


def _argmin_body(z_ref, zsq_ref, esq_ref, e_ref, mask_ref,
                 idx_ref, se_ref, ms_ref):
    zb = z_ref[...]
    zsq = zsq_ref[...]

    def chunk(k, carry):
        run_min, run_idx, run_exact = carry
        e_chunk = e_ref[pl.ds(k * KB, KB), :]
        esq = esq_ref[:, pl.ds(k * KB, KB)]
        mm = lax.dot_general(zb, e_chunk, (((1,), (1,)), ((), ())),
                             preferred_element_type=jnp.float32)
        d = (zsq + esq) - 2.0 * mm
        m = jnp.min(d, axis=1, keepdims=True)
        col = lax.broadcasted_iota(jnp.int32, (TB, KB), 1) + k * KB
        li = jnp.min(jnp.where(d == m, col, NCODE), axis=1, keepdims=True)
        better = m < run_min
        m_carry = m.astype(jnp.bfloat16).astype(jnp.float32)
        return (jnp.where(better, m_carry, run_min),
                jnp.where(better, li, run_idx),
                jnp.where(better, m, run_exact))

    init = (jnp.full((TB, 1), jnp.inf, jnp.float32),
            jnp.zeros((TB, 1), jnp.int32),
            jnp.full((TB, 1), jnp.inf, jnp.float32))
    run_min, run_idx, run_exact = lax.fori_loop(0, NKB, chunk, init)
    idx_ref[...] = run_idx
    mb = mask_ref[...]
    i = pl.program_id(0)
    se_ref[pl.ds(i, 1), :] = jnp.sum(run_exact * mb).reshape(1, 1)
    ms_ref[pl.ds(i, 1), :] = jnp.sum(mb).reshape(1, 1)


def _argmin_call(z_flat, z_sq, e_sq, embeddings, mask_flat):
    nblk = NTOK // TB
    return pl.pallas_call(
        _argmin_body,
        grid=(nblk,),
        in_specs=[
            pl.BlockSpec((TB, D), lambda i: (i, 0)),
            pl.BlockSpec((TB, 1), lambda i: (i, 0)),
            pl.BlockSpec((1, NCODE), lambda i: (0, 0)),
            pl.BlockSpec((NCODE, D), lambda i: (0, 0)),
            pl.BlockSpec((TB, 1), lambda i: (i, 0)),
        ],
        out_specs=[
            pl.BlockSpec((TB, 1), lambda i: (i, 0)),
            pl.BlockSpec((nblk, 1), lambda i: (0, 0)),
            pl.BlockSpec((nblk, 1), lambda i: (0, 0)),
        ],
        out_shape=[
            jax.ShapeDtypeStruct((NTOK, 1), jnp.int32),
            jax.ShapeDtypeStruct((nblk, 1), jnp.float32),
            jax.ShapeDtypeStruct((nblk, 1), jnp.float32),
        ],
    )(z_flat, z_sq, e_sq, embeddings, mask_flat)


GROW = 128


def _make_sc_gather():
    info = plsc.get_sparse_core_info()
    nw = info.num_cores * info.num_subcores
    b_per_w = NTOK // nw
    nch = b_per_w // 128
    mesh = plsc.VectorSubcoreMesh(core_axis_name="c", subcore_axis_name="s")

    @functools.partial(
        pl.kernel, mesh=mesh,
        out_type=jax.ShapeDtypeStruct((NTOK, GROW), jnp.float32),
        scratch_types=[
            pltpu.VMEM((nch, 128), jnp.int32),
            pltpu.VMEM((b_per_w, GROW), jnp.float32),
            pltpu.SemaphoreType.DMA,
        ],
    )
    def gather(table_hbm, idx_hbm, out_hbm, idx_v, rows_v, sem):
        wid = lax.axis_index("s") * info.num_cores + lax.axis_index("c")
        base = wid * b_per_w
        pltpu.sync_copy(idx_hbm.at[pl.ds(wid * nch, nch)], idx_v)
        for c in range(nch):
            pltpu.async_copy(
                table_hbm.at[idx_v.at[c]],
                rows_v.at[pl.ds(c * 128, 128)], sem).wait()
        pltpu.sync_copy(rows_v, out_hbm.at[pl.ds(base, b_per_w)])

    return gather


def kernel(z, mask, embeddings):
    orig_shape = z.shape
    z_flat = z.reshape(-1, z.shape[-1])
    z_sq = jnp.sum(z_flat ** 2, axis=1, keepdims=True)
    e_sq = jnp.sum(embeddings ** 2, axis=1).reshape(1, NCODE)
    mask_flat = mask.reshape(NTOK, 1)

    idx2d, se_parts, ms_parts = _argmin_call(
        z_flat, z_sq, e_sq, embeddings, mask_flat)
    idx = idx2d.reshape(NTOK)

    e_pad = jnp.pad(embeddings, ((0, 0), (0, GROW - D)))
    rows = _make_sc_gather()(e_pad, idx.reshape(NTOK // 128, 128))
    quantized = rows[:, :D].reshape(orig_shape)
    quantized_st = z + (quantized - z)

    mse = (jnp.sum(se_parts) / 32.0) / jnp.maximum(jnp.sum(ms_parts), 1.0)
    loss = mse + 0.25 * mse

    return quantized_st, idx.reshape(orig_shape[:-1]), loss
